# initial kernel scaffold (unmeasured)
import jax
import jax.numpy as jnp
from jax import lax
from jax.experimental import pallas as pl
from jax.experimental.pallas import tpu as pltpu


def kernel(
    x,
):
    def body(*refs):
        pass

    out_shape = jax.ShapeDtypeStruct(..., jnp.float32)
    return pl.pallas_call(body, out_shape=out_shape)(...)


# baseline (device time: 16994 ns/iter reference)
import jax
import jax.numpy as jnp
from jax import lax
from jax.experimental import pallas as pl
from jax.experimental.pallas import tpu as pltpu


def kernel(x):
    m, n = x.shape

    def body(x_ref, out_ref, comm_ref, send_sem, recv_sem):
        my_x = lax.axis_index("x")
        my_y = lax.axis_index("y")
        my_z = lax.axis_index("z")
        nbr = (my_x, 1 - my_y, my_z)

        barrier_sem = pltpu.get_barrier_semaphore()
        pl.semaphore_signal(
            barrier_sem, inc=1, device_id=nbr,
            device_id_type=pl.DeviceIdType.MESH,
        )
        pl.semaphore_wait(barrier_sem, 1)

        rdma = pltpu.make_async_remote_copy(
            src_ref=x_ref,
            dst_ref=comm_ref,
            send_sem=send_sem,
            recv_sem=recv_sem,
            device_id=nbr,
            device_id_type=pl.DeviceIdType.MESH,
        )
        rdma.start()
        rdma.wait()

        out_ref[:, :] = x_ref[:, :] + comm_ref[:, :]

    return pl.pallas_call(
        body,
        out_shape=jax.ShapeDtypeStruct((m, n), x.dtype),
        in_specs=[pl.BlockSpec(memory_space=pltpu.VMEM)],
        out_specs=pl.BlockSpec(memory_space=pltpu.VMEM),
        scratch_shapes=[
            pltpu.VMEM((m, n), x.dtype),
            pltpu.SemaphoreType.DMA,
            pltpu.SemaphoreType.DMA,
        ],
        compiler_params=pltpu.CompilerParams(collective_id=0),
    )(x)


# device time: 14798 ns/iter; 1.1484x vs baseline; 1.1484x over previous
import jax
import jax.numpy as jnp
from jax import lax
from jax.experimental import pallas as pl
from jax.experimental.pallas import tpu as pltpu

CHUNKS = 8


def kernel(x):
    m, n = x.shape
    half = m // 2
    rows = half // CHUNKS

    def body(x_ref, out_ref, comm_ref, send_y, recv_y, send_x, recv_x):
        my_x = lax.axis_index("x")
        my_y = lax.axis_index("y")
        my_z = lax.axis_index("z")
        y_nbr = (my_x, 1 - my_y, my_z)
        x_nbr = (1 - my_x, my_y, my_z)

        barrier_sem = pltpu.get_barrier_semaphore()
        for nbr in (y_nbr, x_nbr):
            pl.semaphore_signal(
                barrier_sem, inc=1, device_id=nbr,
                device_id_type=pl.DeviceIdType.MESH,
            )
        pl.semaphore_wait(barrier_sem, 2)

        base = my_x * half

        y_rdmas = []
        for c in range(CHUNKS):
            r = pltpu.make_async_remote_copy(
                src_ref=x_ref.at[pl.ds(base + c * rows, rows)],
                dst_ref=comm_ref.at[pl.ds(c * rows, rows)],
                send_sem=send_y.at[c],
                recv_sem=recv_y.at[c],
                device_id=y_nbr,
                device_id_type=pl.DeviceIdType.MESH,
            )
            r.start()
            y_rdmas.append(r)

        x_rdmas = []
        for c in range(CHUNKS):
            y_rdmas[c].wait_recv()
            sl = pl.ds(base + c * rows, rows)
            out_ref[sl, :] = x_ref[sl, :] + comm_ref[pl.ds(c * rows, rows), :]
            r = pltpu.make_async_remote_copy(
                src_ref=out_ref.at[sl],
                dst_ref=out_ref.at[sl],
                send_sem=send_x.at[c],
                recv_sem=recv_x.at[c],
                device_id=x_nbr,
                device_id_type=pl.DeviceIdType.MESH,
            )
            r.start()
            x_rdmas.append(r)

        for c in range(CHUNKS):
            x_rdmas[c].wait_recv()
        for c in range(CHUNKS):
            y_rdmas[c].wait_send()
            x_rdmas[c].wait_send()

    return pl.pallas_call(
        body,
        out_shape=jax.ShapeDtypeStruct((m, n), x.dtype),
        in_specs=[pl.BlockSpec(memory_space=pltpu.VMEM)],
        out_specs=pl.BlockSpec(memory_space=pltpu.VMEM),
        scratch_shapes=[
            pltpu.VMEM((half, n), x.dtype),
            pltpu.SemaphoreType.DMA((CHUNKS,)),
            pltpu.SemaphoreType.DMA((CHUNKS,)),
            pltpu.SemaphoreType.DMA((CHUNKS,)),
            pltpu.SemaphoreType.DMA((CHUNKS,)),
        ],
        compiler_params=pltpu.CompilerParams(collective_id=0),
    )(x)


# device time: 14353 ns/iter; 1.1840x vs baseline; 1.0310x over previous
import os

import jax
import jax.numpy as jnp
from jax import lax
from jax.experimental import pallas as pl
from jax.experimental.pallas import tpu as pltpu

SUB = int(os.environ.get("AR_SUB", "4"))
N_SLICES = 8
KEEP_OFFS = (0, 2, 5)
LEFT_OFFS = (2, 5)
FROM_LEFT_OFFS = (7, 1, 4)
FROM_RIGHT_OFFS = (3, 6)


def kernel(x):
    m, n = x.shape
    srows = m // N_SLICES
    crows = srows // SUB
    n_keep = len(KEEP_OFFS) * SUB
    n_left = len(LEFT_OFFS) * SUB

    def body(
        x_ref, out_ref, comm_ref,
        send_y, recv_y, send_r, recv_l, send_l, recv_r,
    ):
        my_x = lax.axis_index("x")
        my_y = lax.axis_index("y")
        my_z = lax.axis_index("z")
        y_nbr = (my_x, 1 - my_y, my_z)

        p = my_z + my_x * (7 - 2 * my_z)

        def ring_coords(rp):
            on_top = rp <= 3
            return (
                jnp.where(on_top, 0, 1),
                my_y,
                jnp.where(on_top, rp, 7 - rp),
            )

        right = ring_coords((p + 1) % 8)
        left = ring_coords((p + 7) % 8)

        barrier_sem = pltpu.get_barrier_semaphore()
        for nbr in (y_nbr, right, left):
            pl.semaphore_signal(
                barrier_sem, inc=1, device_id=nbr,
                device_id_type=pl.DeviceIdType.MESH,
            )
        pl.semaphore_wait(barrier_sem, 3)

        y_rdmas = []
        for i, off in enumerate(KEEP_OFFS):
            start = ((p + off) % 8) * srows
            for s in range(SUB):
                k = i * SUB + s
                r = pltpu.make_async_remote_copy(
                    src_ref=x_ref.at[pl.ds(start + s * crows, crows)],
                    dst_ref=comm_ref.at[pl.ds(k * crows, crows)],
                    send_sem=send_y.at[k],
                    recv_sem=recv_y.at[k],
                    device_id=y_nbr,
                    device_id_type=pl.DeviceIdType.MESH,
                )
                r.start()
                y_rdmas.append(r)

        fwd_rdmas = []
        for i, off in enumerate(KEEP_OFFS):
            start = ((p + off) % 8) * srows
            for s in range(SUB):
                k = i * SUB + s
                y_rdmas[k].wait_recv()
                sl = pl.ds(start + s * crows, crows)
                out_ref[sl, :] = (
                    x_ref[sl, :] + comm_ref[pl.ds(k * crows, crows), :]
                )
                r = pltpu.make_async_remote_copy(
                    src_ref=out_ref.at[sl],
                    dst_ref=out_ref.at[sl],
                    send_sem=send_r.at[k],
                    recv_sem=recv_l.at[k],
                    device_id=right,
                    device_id_type=pl.DeviceIdType.MESH,
                )
                r.start()
                fwd_rdmas.append(r)
                if i >= 1:
                    kl = (i - 1) * SUB + s
                    r = pltpu.make_async_remote_copy(
                        src_ref=out_ref.at[sl],
                        dst_ref=out_ref.at[sl],
                        send_sem=send_l.at[kl],
                        recv_sem=recv_r.at[kl],
                        device_id=left,
                        device_id_type=pl.DeviceIdType.MESH,
                    )
                    r.start()
                    fwd_rdmas.append(r)

        for i, off in enumerate(FROM_LEFT_OFFS):
            start = ((p + off) % 8) * srows
            for s in range(SUB):
                sl = pl.ds(start + s * crows, crows)
                pltpu.make_async_remote_copy(
                    src_ref=out_ref.at[sl],
                    dst_ref=out_ref.at[sl],
                    send_sem=send_y.at[0],
                    recv_sem=recv_l.at[i * SUB + s],
                    device_id=left,
                    device_id_type=pl.DeviceIdType.MESH,
                ).wait_recv()
        for i, off in enumerate(FROM_RIGHT_OFFS):
            start = ((p + off) % 8) * srows
            for s in range(SUB):
                sl = pl.ds(start + s * crows, crows)
                pltpu.make_async_remote_copy(
                    src_ref=out_ref.at[sl],
                    dst_ref=out_ref.at[sl],
                    send_sem=send_y.at[0],
                    recv_sem=recv_r.at[i * SUB + s],
                    device_id=right,
                    device_id_type=pl.DeviceIdType.MESH,
                ).wait_recv()

        for r in y_rdmas:
            r.wait_send()
        for r in fwd_rdmas:
            r.wait_send()

    return pl.pallas_call(
        body,
        out_shape=jax.ShapeDtypeStruct((m, n), x.dtype),
        in_specs=[pl.BlockSpec(memory_space=pltpu.VMEM)],
        out_specs=pl.BlockSpec(memory_space=pltpu.VMEM),
        scratch_shapes=[
            pltpu.VMEM((len(KEEP_OFFS) * srows, n), x.dtype),
            pltpu.SemaphoreType.DMA((n_keep,)),
            pltpu.SemaphoreType.DMA((n_keep,)),
            pltpu.SemaphoreType.DMA((n_keep,)),
            pltpu.SemaphoreType.DMA((n_keep,)),
            pltpu.SemaphoreType.DMA((n_left,)),
            pltpu.SemaphoreType.DMA((n_left,)),
        ],
        compiler_params=pltpu.CompilerParams(collective_id=0),
    )(x)
